# Initial kernel scaffold; baseline (speedup 1.0000x reference)
import jax
import jax.numpy as jnp
from jax.experimental import pallas as pl


def kernel(inputs, segment_ids, gain, bias):
    raise NotImplementedError("write your pallas kernel here")


# speedup vs baseline: 5.0582x; 5.0582x over previous
import functools

import jax
import jax.numpy as jnp
from jax import lax
from jax.experimental import pallas as pl
from jax.experimental.pallas import tpu as pltpu
from jax.experimental.pallas import tpu_sc as plsc

N = 320000
D = 128
S = 10000
SEG_PAD = 10240
EPS = 1e-05

NC = 2
NS = 16
NW = NC * NS

GP = 80
G_PAD = GP * NW
N_PAD = G_PAD * 128
RW = N // NW
DUMMY_SEG = SEG_PAD - 1
SEG_SL = SEG_PAD // NS

_BR = 1600


def _rowstats_body(x_ref, s_ref, s2_ref):
    x = x_ref[...]
    s_ref[...] = jnp.sum(x, axis=1, keepdims=True)
    s2_ref[...] = jnp.sum(x * x, axis=1, keepdims=True)


def _row_stats(x):
    return pl.pallas_call(
        _rowstats_body,
        grid=(N // _BR,),
        in_specs=[pl.BlockSpec((_BR, D), lambda i: (i, 0))],
        out_specs=[pl.BlockSpec((_BR, 1), lambda i: (i, 0)),
                   pl.BlockSpec((_BR, 1), lambda i: (i, 0))],
        out_shape=[jax.ShapeDtypeStruct((N, 1), jnp.float32),
                   jax.ShapeDtypeStruct((N, 1), jnp.float32)],
        compiler_params=pltpu.CompilerParams(
            dimension_semantics=("parallel",)),
    )(x)


def _sc_segsum(ids2d, rs, rs2):
    mesh = plsc.VectorSubcoreMesh(core_axis_name="c", subcore_axis_name="s")

    @functools.partial(
        pl.kernel,
        out_type=jax.ShapeDtypeStruct((NC * 3 * SEG_PAD,), jnp.float32),
        mesh=mesh,
        scratch_types=[
            pltpu.VMEM((GP, 128), jnp.int32),
            pltpu.VMEM((GP * 128,), jnp.float32),
            pltpu.VMEM((GP * 128,), jnp.float32),
            pltpu.VMEM((128,), jnp.float32),
            pltpu.VMEM((SEG_SL,), jnp.float32),
            pltpu.VMEM_SHARED((SEG_PAD,), jnp.float32),
            pltpu.VMEM_SHARED((SEG_PAD,), jnp.float32),
            pltpu.VMEM_SHARED((SEG_PAD,), jnp.float32),
        ],
    )
    def k(ids_hbm, rs_hbm, rs2_hbm, out_hbm,
          idx_v, rs_v, rs2_v, ones_v, z_v, acc_s, acc_s2, acc_c):
        c = lax.axis_index("c")
        s = lax.axis_index("s")
        w = s * NC + c

        zv = jnp.zeros((16,), jnp.float32)
        for i in range(SEG_SL // 16):
            z_v[pl.ds(i * 16, 16)] = zv
        ov = jnp.ones((16,), jnp.float32)
        for i in range(128 // 16):
            ones_v[pl.ds(i * 16, 16)] = ov

        sl = pl.ds(s * SEG_SL, SEG_SL)
        pltpu.sync_copy(z_v, acc_s.at[sl])
        pltpu.sync_copy(z_v, acc_s2.at[sl])
        pltpu.sync_copy(z_v, acc_c.at[sl])

        pltpu.sync_copy(ids_hbm.at[pl.ds(w * GP, GP)], idx_v)
        pltpu.sync_copy(rs_hbm.at[pl.ds(w * (GP * 128), GP * 128)], rs_v)
        pltpu.sync_copy(rs2_hbm.at[pl.ds(w * (GP * 128), GP * 128)], rs2_v)

        plsc.subcore_barrier()

        def body(j, carry):
            idx = idx_v.at[j]
            pltpu.sync_copy(rs_v.at[pl.ds(j * 128, 128)], acc_s.at[idx],
                            add=True)
            pltpu.sync_copy(rs2_v.at[pl.ds(j * 128, 128)], acc_s2.at[idx],
                            add=True)
            pltpu.sync_copy(ones_v, acc_c.at[idx], add=True)
            return carry

        lax.fori_loop(0, GP, body, 0)

        plsc.subcore_barrier()

        base = c * (3 * SEG_PAD)
        off = s * SEG_SL
        pltpu.sync_copy(acc_s.at[sl], out_hbm.at[pl.ds(base + off, SEG_SL)])
        pltpu.sync_copy(acc_s2.at[sl],
                        out_hbm.at[pl.ds(base + SEG_PAD + off, SEG_SL)])
        pltpu.sync_copy(acc_c.at[sl],
                        out_hbm.at[pl.ds(base + 2 * SEG_PAD + off, SEG_SL)])

    return k(ids2d, rs, rs2)


def _sc_gather(partials, ids):
    mesh = plsc.VectorSubcoreMesh(core_axis_name="c", subcore_axis_name="s")

    @functools.partial(
        pl.kernel,
        out_type=[jax.ShapeDtypeStruct((N,), jnp.float32),
                  jax.ShapeDtypeStruct((N,), jnp.float32)],
        mesh=mesh,
        compiler_params=pltpu.CompilerParams(needs_layout_passes=False),
        scratch_types=[
            pltpu.VMEM((NC * 3 * SEG_PAD,), jnp.float32),
            pltpu.VMEM((SEG_PAD,), jnp.float32),
            pltpu.VMEM((SEG_PAD,), jnp.float32),
            pltpu.VMEM((RW,), jnp.int32),
            pltpu.VMEM((RW,), jnp.float32),
            pltpu.VMEM((RW,), jnp.float32),
        ],
    )
    def k(p_hbm, ids_hbm, m_hbm, v_hbm, p_v, mt_v, vt_v, ids_v, m_v, v_v):
        c = lax.axis_index("c")
        s = lax.axis_index("s")
        w = s * NC + c

        pltpu.sync_copy(p_hbm, p_v)
        pltpu.sync_copy(ids_hbm.at[pl.ds(w * RW, RW)], ids_v)

        def tbody(i, carry):
            b = i * 16
            ssum = p_v[pl.ds(b, 16)] + p_v[pl.ds(3 * SEG_PAD + b, 16)]
            ssq = (p_v[pl.ds(SEG_PAD + b, 16)]
                   + p_v[pl.ds(4 * SEG_PAD + b, 16)])
            cnt = (p_v[pl.ds(2 * SEG_PAD + b, 16)]
                   + p_v[pl.ds(5 * SEG_PAD + b, 16)])
            div = jnp.maximum(cnt * float(D), 1.0)
            mean = ssum / div
            var = ssq / div - mean * mean
            mt_v[pl.ds(b, 16)] = mean
            vt_v[pl.ds(b, 16)] = var
            return carry

        lax.fori_loop(0, SEG_PAD // 16, tbody, 0)

        def gbody(i, carry):
            b = i * 16
            idx = ids_v[pl.ds(b, 16)]
            m_v[pl.ds(b, 16)] = plsc.load_gather(mt_v, [idx])
            v_v[pl.ds(b, 16)] = plsc.load_gather(vt_v, [idx])
            return carry

        lax.fori_loop(0, RW // 16, gbody, 0)

        pltpu.sync_copy(m_v, m_hbm.at[pl.ds(w * RW, RW)])
        pltpu.sync_copy(v_v, v_hbm.at[pl.ds(w * RW, RW)])

    return k(partials, ids)


def _norm_body(x_ref, m_ref, v_ref, g_ref, b_ref, o_ref):
    x = x_ref[...]
    r = lax.rsqrt(v_ref[...] + EPS)
    o_ref[...] = (x - m_ref[...]) * r * g_ref[...] + b_ref[...]


def _normalize(x, m, v, gain, bias):
    return pl.pallas_call(
        _norm_body,
        grid=(N // _BR,),
        in_specs=[pl.BlockSpec((_BR, D), lambda i: (i, 0)),
                  pl.BlockSpec((_BR, 1), lambda i: (i, 0)),
                  pl.BlockSpec((_BR, 1), lambda i: (i, 0)),
                  pl.BlockSpec((1, D), lambda i: (0, 0)),
                  pl.BlockSpec((1, D), lambda i: (0, 0))],
        out_specs=pl.BlockSpec((_BR, D), lambda i: (i, 0)),
        out_shape=jax.ShapeDtypeStruct((N, D), jnp.float32),
        compiler_params=pltpu.CompilerParams(
            dimension_semantics=("parallel",)),
    )(x, m, v, gain, bias)


def kernel(inputs, segment_ids, gain, bias):
    x = inputs.astype(jnp.float32)
    ids = segment_ids.astype(jnp.int32)

    rs, rs2 = _row_stats(x)

    zpad = jnp.zeros((N_PAD - N,), jnp.float32)
    ids_p = jnp.concatenate(
        [ids, jnp.full((N_PAD - N,), DUMMY_SEG, jnp.int32)]).reshape(G_PAD, 128)
    rs_p = jnp.concatenate([rs.reshape(N), zpad])
    rs2_p = jnp.concatenate([rs2.reshape(N), zpad])

    partials = _sc_segsum(ids_p, rs_p, rs2_p)
    m_r, v_r = _sc_gather(partials, ids)

    return _normalize(x, m_r.reshape(N, 1), v_r.reshape(N, 1),
                      gain.reshape(1, D), bias.reshape(1, D))
